# baseline (speedup 1.0000x reference)
import functools

import jax
import jax.numpy as jnp
from jax import lax
from jax.experimental import pallas as pl
from jax.experimental.pallas import tpu as pltpu
from jax.experimental.pallas import tpu_sc as plsc

EMB = 64
N = 25000
E = 800000
IW = 128
IR = 6400
EP = IR * IW
NPAD = EP - E
NACC = 25600
IB = 8
GROWS = IB * IW
NFULL = IR // 2 // 16 // IB
NB = 1000
EB = 4096

_f32 = jnp.float32


def _ln(x, g, b):
    m = jnp.mean(x, axis=-1, keepdims=True)
    v = jnp.mean((x - m) ** 2, axis=-1, keepdims=True)
    return (x - m) * lax.rsqrt(v + 1e-5) * g + b


def _embed_body(f_ref, g_ref, b_ref, w1_ref, b1_ref, w2_ref, b2_ref, o_ref):
    h = _ln(f_ref[...], g_ref[...], b_ref[...])
    h = jnp.maximum(h @ w1_ref[...] + b1_ref[...], 0.0)
    o_ref[...] = jnp.maximum(h @ w2_ref[...] + b2_ref[...], 0.0)


def _embed(feat, g, b, w1, b1, w2, b2):
    f = feat.shape[1]
    fixed = lambda *blk: pl.BlockSpec(blk, lambda i: (0,) * len(blk))
    return pl.pallas_call(
        _embed_body,
        grid=(N // NB,),
        in_specs=[
            pl.BlockSpec((NB, f), lambda i: (i, 0)),
            fixed(1, f), fixed(1, f), fixed(f, EMB), fixed(1, EMB),
            fixed(EMB, EMB), fixed(1, EMB),
        ],
        out_specs=pl.BlockSpec((NB, EMB), lambda i: (i, 0)),
        out_shape=jax.ShapeDtypeStruct((N, EMB), _f32),
    )(feat, g.reshape(1, f), b.reshape(1, f), w1, b1.reshape(1, EMB),
      w2, b2.reshape(1, EMB))


def _pre_body(r_ref, l_ref, wl_ref, bl_ref, eb_ref, we_ref, wr_ref,
              rw_ref, lw_ref):
    bias = bl_ref[...] + eb_ref[...] * we_ref[...]
    rw_ref[...] = r_ref[...] @ wl_ref[...] + bias
    lw_ref[...] = l_ref[...] @ wr_ref[...]


def _node_pre(right, left, pp, ee_b):
    fixed = lambda *blk: pl.BlockSpec(blk, lambda i: (0,) * len(blk))
    node = pl.BlockSpec((NB, EMB), lambda i: (i, 0))
    return pl.pallas_call(
        _pre_body,
        grid=(N // NB,),
        in_specs=[node, node, fixed(EMB, EMB), fixed(1, EMB), fixed(1, 1),
                  fixed(1, EMB), fixed(EMB, EMB)],
        out_specs=[node, node],
        out_shape=[jax.ShapeDtypeStruct((N, EMB), _f32)] * 2,
    )(right, left, pp['Wl'], pp['bl'].reshape(1, EMB), ee_b.reshape(1, 1),
      pp['We'].reshape(1, EMB), pp['Wr'])


def _edgeln_body(s_ref, fg_ref, fb_ref, o_ref):
    o_ref[...] = jnp.maximum(_ln(s_ref[...], fg_ref[...], fb_ref[...]), 0.0)


def _edge_ln(s, fg, fb):
    fixed = lambda *blk: pl.BlockSpec(blk, lambda i: (0,) * len(blk))
    edge = pl.BlockSpec((EB, EMB), lambda i: (i, 0))
    return pl.pallas_call(
        _edgeln_body,
        grid=(EP // EB,),
        in_specs=[edge, fixed(1, EMB), fixed(1, EMB)],
        out_specs=edge,
        out_shape=jax.ShapeDtypeStruct((EP, EMB), _f32),
    )(s, fg.reshape(1, EMB), fb.reshape(1, EMB))


def _post_body(a0_ref, a1_ref, d0_ref, d1_ref, r_ref, wf_ref, bf_ref,
               pg_ref, pb_ref, w1a_ref, w1b_ref, b1_ref, w2_ref, b2_ref,
               o_ref):
    deg = d0_ref[0][:, :1] + d1_ref[0][:, :1]
    agg = (a0_ref[0] + a1_ref[0]) @ wf_ref[...] + deg * bf_ref[...]
    h = _ln(agg, pg_ref[...], pb_ref[...])
    t = jnp.maximum(h @ w1a_ref[...] + r_ref[...] @ w1b_ref[...] + b1_ref[...],
                    0.0)
    o_ref[...] = t @ w2_ref[...] + b2_ref[...]


def _node_post(acc, deg, right, pp):
    fixed = lambda *blk: pl.BlockSpec(blk, lambda i: (0,) * len(blk))
    node = pl.BlockSpec((NB, EMB), lambda i: (i, 0))
    return pl.pallas_call(
        _post_body,
        grid=(N // NB,),
        in_specs=[
            pl.BlockSpec((1, NB, EMB), lambda i: (0, i, 0)),
            pl.BlockSpec((1, NB, EMB), lambda i: (1, i, 0)),
            pl.BlockSpec((1, NB, 16), lambda i: (0, i, 0)),
            pl.BlockSpec((1, NB, 16), lambda i: (1, i, 0)),
            node,
            fixed(EMB, EMB), fixed(1, EMB), fixed(1, EMB), fixed(1, EMB),
            fixed(EMB, EMB), fixed(EMB, EMB), fixed(1, EMB),
            fixed(EMB, EMB), fixed(1, EMB),
        ],
        out_specs=node,
        out_shape=jax.ShapeDtypeStruct((N, EMB), _f32),
    )(acc, acc, deg, deg, right,
      pp['Wf'], pp['bf'].reshape(1, EMB), pp['pg'].reshape(1, EMB),
      pp['pb'].reshape(1, EMB), pp['Wo1'][:EMB], pp['Wo1'][EMB:],
      pp['bo1'].reshape(1, EMB), pp['Wo2'], pp['bo2'].reshape(1, EMB))


def _mlp_body(x_ref, w1_ref, b1_ref, w2_ref, b2_ref, w3_ref, b3_ref, o_ref):
    h = jnp.maximum(x_ref[...] @ w1_ref[...] + b1_ref[...], 0.0)
    h = jnp.maximum(h @ w2_ref[...] + b2_ref[...], 0.0)
    o_ref[...] = h @ w3_ref[...] + b3_ref[...]


def _mlp3(x, pp):
    fixed = lambda *blk: pl.BlockSpec(blk, lambda i: (0,) * len(blk))
    return pl.pallas_call(
        _mlp_body,
        grid=(N // NB,),
        in_specs=[pl.BlockSpec((NB, EMB), lambda i: (i, 0)),
                  fixed(EMB, EMB), fixed(1, EMB), fixed(EMB, EMB),
                  fixed(1, EMB), fixed(EMB, 16), fixed(1, 16)],
        out_specs=pl.BlockSpec((NB, 16), lambda i: (i, 0)),
        out_shape=jax.ShapeDtypeStruct((N, 16), _f32),
    )(x, pp['W1'], pp['b1'].reshape(1, EMB), pp['W2'],
      pp['b2'].reshape(1, EMB), pp['W3'], pp['b3'].reshape(1, 16))


_SC_MESH = plsc.VectorSubcoreMesh(core_axis_name="c", subcore_axis_name="s")


CIB = 2
CROWS = CIB * IW
NCH = IR // 2 // 16 // CIB


@functools.partial(
    pl.kernel,
    out_type=jax.ShapeDtypeStruct((EP, EMB), _f32),
    mesh=_SC_MESH,
    compiler_params=pltpu.CompilerParams(use_tc_tiling_on_sc=False,
                                         needs_layout_passes=False),
    scratch_types=[
        pltpu.VMEM((CIB, IW), jnp.int32),
        pltpu.VMEM((CIB, IW), jnp.int32),
        pltpu.VMEM((CIB, IW), jnp.int32),
        pltpu.VMEM((CIB, IW), jnp.int32),
        pltpu.VMEM((CROWS, EMB), _f32),
        pltpu.VMEM((CROWS, EMB), _f32),
        pltpu.VMEM((CROWS, EMB), _f32),
        pltpu.VMEM((CROWS, EMB), _f32),
        pltpu.VMEM((2, EMB), _f32),
        pltpu.SemaphoreType.DMA,
        pltpu.SemaphoreType.DMA,
        pltpu.SemaphoreType.DMA,
        pltpu.SemaphoreType.DMA,
    ],
)
def _edge_gather(ta_hbm, tb_hbm, ia_hbm, ib_hbm, fgfb_hbm, o_hbm,
                 ixa0, ixa1, ixb0, ixb1, ra0, ra1, rb0, rb1, fgfb,
                 sg0, sg1, ss0, ss1):
    cid = lax.axis_index("c")
    sid = lax.axis_index("s")
    r0 = (cid * 16 + sid) * (NCH * CIB)
    pltpu.sync_copy(fgfb_hbm, fgfb)
    fgl = [fgfb[0, pl.ds(k * 16, 16)] for k in range(4)]
    fbl = [fgfb[1, pl.ds(k * 16, 16)] for k in range(4)]

    def load_idx(rb, ixa, ixb):
        pltpu.sync_copy(ia_hbm.at[pl.ds(rb, CIB)], ixa)
        pltpu.sync_copy(ib_hbm.at[pl.ds(rb, CIB)], ixb)

    def issue_gathers(ixa, ixb, ra, rbuf, sem):
        for j in range(CIB):
            pltpu.async_copy(ta_hbm.at[ixa.at[j]],
                             ra.at[pl.ds(j * IW, IW)], sem)
            pltpu.async_copy(tb_hbm.at[ixb.at[j]],
                             rbuf.at[pl.ds(j * IW, IW)], sem)

    def drain_gathers(ra, rbuf, sem):
        pltpu.make_async_copy(ta_hbm.at[pl.ds(0, CROWS)], ra, sem).wait()
        pltpu.make_async_copy(ta_hbm.at[pl.ds(0, CROWS)], rbuf, sem).wait()

    def add_rows(ra, rbuf):
        inv = 1.0 / EMB

        def abody(r2, carry):
            for u in range(2):
                r = r2 * 2 + u
                s = [ra[r, pl.ds(k * 16, 16)] + rbuf[r, pl.ds(k * 16, 16)]
                     for k in range(4)]
                t = (s[0] + s[1]) + (s[2] + s[3])
                sm = jnp.sum(t) * inv
                mean_v = jnp.broadcast_to(sm, (16,))
                d = [s[k] - mean_v for k in range(4)]
                q = (d[0] * d[0] + d[1] * d[1]) + (d[2] * d[2] + d[3] * d[3])
                var_v = jnp.broadcast_to(jnp.sum(q) * inv + 1e-5, (16,))
                seed = jnp.int32(0x5F3759DF) - (
                    plsc.bitcast(var_v, jnp.int32) >> 1)
                y = plsc.bitcast(seed, _f32)
                xh = var_v * 0.5
                y = y * (1.5 - xh * y * y)
                y = y * (1.5 - xh * y * y)
                y = y * (1.5 - xh * y * y)
                for k in range(4):
                    o = d[k] * (y * fgl[k]) + fbl[k]
                    ra[r, pl.ds(k * 16, 16)] = jnp.maximum(o, 0.0)
            return carry
        lax.fori_loop(0, CROWS // 2, abody, 0)

    def drain_store(ra, sem):
        pltpu.make_async_copy(ra, o_hbm.at[pl.ds(0, CROWS)], sem).wait()

    load_idx(r0, ixa0, ixb0)
    issue_gathers(ixa0, ixb0, ra0, rb0, sg0)

    def body(i, carry):
        c0 = r0 + 2 * i * CIB
        load_idx(c0 + CIB, ixa1, ixb1)

        @pl.when(i >= 1)
        def _():
            drain_store(ra1, ss1)
        issue_gathers(ixa1, ixb1, ra1, rb1, sg1)
        drain_gathers(ra0, rb0, sg0)
        add_rows(ra0, rb0)
        pltpu.async_copy(ra0, o_hbm.at[pl.ds(c0 * IW, CROWS)], ss0)

        @pl.when(i < NCH // 2 - 1)
        def _():
            load_idx(c0 + 2 * CIB, ixa0, ixb0)
            drain_store(ra0, ss0)
            issue_gathers(ixa0, ixb0, ra0, rb0, sg0)
        drain_gathers(ra1, rb1, sg1)
        add_rows(ra1, rb1)
        pltpu.async_copy(ra1, o_hbm.at[pl.ds((c0 + CIB) * IW, CROWS)], ss1)
        return carry

    lax.fori_loop(0, NCH // 2, body, 0)
    drain_store(ra0, ss0)
    drain_store(ra1, ss1)


NSCH = IR // 2 // 16


@functools.partial(
    pl.kernel,
    out_type=jax.ShapeDtypeStruct((2, N, EMB), _f32),
    mesh=_SC_MESH,
    compiler_params=pltpu.CompilerParams(use_tc_tiling_on_sc=False),
    scratch_types=[
        pltpu.VMEM((1, IW), jnp.int32),
        pltpu.VMEM((1, IW), jnp.int32),
        pltpu.VMEM((IW, EMB), _f32),
        pltpu.VMEM((IW, EMB), _f32),
        pltpu.VMEM_SHARED((NACC, EMB), _f32),
        pltpu.SemaphoreType.DMA,
        pltpu.SemaphoreType.DMA,
    ],
)
def _edge_scatter(x_hbm, idx_hbm, z_hbm, out_hbm,
                  ix0, ix1, x0, x1, acc, sx0, sx1):
    cid = lax.axis_index("c")
    sid = lax.axis_index("s")

    pltpu.sync_copy(z_hbm, x0)
    zbase = lax.min(sid * 1600, NACC - 14 * IW)
    for j in range(14):
        pltpu.sync_copy(x0, acc.at[pl.ds(zbase + j * IW, IW)])
    plsc.subcore_barrier()

    r0 = (cid * 16 + sid) * NSCH

    def load(r, ix, xb, sem):
        pltpu.sync_copy(idx_hbm.at[pl.ds(r, 1)], ix)
        pltpu.async_copy(x_hbm.at[pl.ds(r * IW, IW)], xb, sem)

    def scat(ix, xb, sem):
        pltpu.make_async_copy(x_hbm.at[pl.ds(0, IW)], xb, sem).wait()
        pltpu.sync_copy(xb, acc.at[ix.at[0]], add=True)

    load(r0, ix0, x0, sx0)

    def body(i, carry):
        c0 = r0 + 2 * i
        load(c0 + 1, ix1, x1, sx1)
        scat(ix0, x0, sx0)

        @pl.when(i < NSCH // 2 - 1)
        def _():
            load(c0 + 2, ix0, x0, sx0)
        scat(ix1, x1, sx1)
        return carry

    lax.fori_loop(0, NSCH // 2, body, 0)
    plsc.subcore_barrier()

    @pl.when(sid == 0)
    def _():
        pltpu.sync_copy(acc.at[pl.ds(0, N)], out_hbm.at[cid])


@functools.partial(
    pl.kernel,
    out_type=[jax.ShapeDtypeStruct((2, N, 16), _f32)] * 2,
    mesh=_SC_MESH,
    compiler_params=pltpu.CompilerParams(use_tc_tiling_on_sc=False),
    scratch_types=[
        pltpu.VMEM((IB, IW), jnp.int32),
        pltpu.VMEM((IB, IW), jnp.int32),
        pltpu.VMEM((IW, 16), _f32),
        pltpu.VMEM_SHARED((NACC, 16), _f32),
        pltpu.VMEM_SHARED((NACC, 16), _f32),
        pltpu.SemaphoreType.DMA,
        pltpu.SemaphoreType.DMA,
    ],
)
def _degree(i0_hbm, i1_hbm, z_hbm, one_hbm, o0_hbm, o1_hbm,
            ix0, ix1, ones, acc0, acc1, sa, sb):
    cid = lax.axis_index("c")
    sid = lax.axis_index("s")

    pltpu.sync_copy(z_hbm, ones)
    zbase = lax.min(sid * 1600, NACC - 13 * IW)
    for j in range(13):
        pltpu.sync_copy(ones, acc0.at[pl.ds(zbase + j * IW, IW)])
        pltpu.sync_copy(ones, acc1.at[pl.ds(zbase + j * IW, IW)])
    plsc.subcore_barrier()
    pltpu.sync_copy(one_hbm, ones)

    r0 = (cid * 16 + sid) * (NFULL * IB)

    def drain(acc, ix, sem):
        for j in range(IB):
            pltpu.make_async_copy(ones, acc.at[ix.at[j]], sem).wait()

    def outer(k, carry):
        rb = r0 + k * IB

        @pl.when(k >= 1)
        def _():
            drain(acc1, ix1, sb)
        pltpu.sync_copy(i0_hbm.at[pl.ds(rb, IB)], ix0)
        for j in range(IB):
            pltpu.async_copy(ones, acc0.at[ix0.at[j]], sa, add=True)
        pltpu.sync_copy(i1_hbm.at[pl.ds(rb, IB)], ix1)
        drain(acc0, ix0, sa)
        for j in range(IB):
            pltpu.async_copy(ones, acc1.at[ix1.at[j]], sb, add=True)
        return carry

    lax.fori_loop(0, NFULL, outer, 0)
    drain(acc1, ix1, sb)
    plsc.subcore_barrier()

    @pl.when(sid == 0)
    def _():
        pltpu.sync_copy(acc0.at[pl.ds(0, N)], o0_hbm.at[cid])
        pltpu.sync_copy(acc1.at[pl.ds(0, N)], o1_hbm.at[cid])


def _gnn_pass(pp, right, left, dst_g, src_g, dst_s, deg, ee_b, zeros_g):
    rw, lw = _node_pre(right, left, pp, ee_b)
    fgfb = jnp.stack([pp['fg'], pp['fb']])
    x = _edge_gather(rw, lw, dst_g, src_g, fgfb)
    acc = _edge_scatter(x, dst_s, zeros_g)
    return _node_post(acc, deg, right, pp)


def kernel(constraint_features, edge_indices, edge_features,
           variable_features, params):
    p = params
    del edge_features

    pad_g = jnp.arange(NPAD, dtype=jnp.int32) % N
    pad_s = N + (jnp.arange(NPAD, dtype=jnp.int32) % (NACC - N))
    g0 = jnp.concatenate([edge_indices[0], pad_g]).reshape(IR, IW)
    g1 = jnp.concatenate([edge_indices[1], pad_g]).reshape(IR, IW)
    s0 = jnp.concatenate([edge_indices[0], pad_s]).reshape(IR, IW)
    s1 = jnp.concatenate([edge_indices[1], pad_s]).reshape(IR, IW)

    zeros_g = jnp.zeros((IW, EMB), _f32)
    zeros_d = jnp.zeros((IW, 16), _f32)
    ones_d = jnp.ones((IW, 16), _f32)

    c = _embed(constraint_features, p['ce_g'], p['ce_b'], p['ce_W1'],
               p['ce_b1'], p['ce_W2'], p['ce_b2'])
    v = _embed(variable_features, p['ve_g'], p['ve_b'], p['ve_W1'],
               p['ve_b1'], p['ve_W2'], p['ve_b2'])
    deg_c, deg_v = _degree(s0, s1, zeros_d, ones_d)

    for _ in range(3):
        c = _gnn_pass(p['vc'], c, v, g0, g1, s0, deg_c, p['ee_b'], zeros_g)
        v = _gnn_pass(p['cv'], v, c, g1, g0, s1, deg_v, p['ee_b'], zeros_g)

    return _mlp3(c, p['co']), _mlp3(v, p['vo'])


# speedup vs baseline: 1.3126x; 1.3126x over previous
import functools

import jax
import jax.numpy as jnp
from jax import lax
from jax.experimental import pallas as pl
from jax.experimental.pallas import tpu as pltpu
from jax.experimental.pallas import tpu_sc as plsc

EMB = 64
N = 25000
E = 800000
IW = 128
IR = 6400
EP = IR * IW
NPAD = EP - E
NACC = 25600
IB = 8
GROWS = IB * IW
NFULL = IR // 2 // 16 // IB
NB = 1000
EB = 4096

_f32 = jnp.float32


def _ln(x, g, b):
    m = jnp.mean(x, axis=-1, keepdims=True)
    v = jnp.mean((x - m) ** 2, axis=-1, keepdims=True)
    return (x - m) * lax.rsqrt(v + 1e-5) * g + b


def _embed_body(f_ref, g_ref, b_ref, w1_ref, b1_ref, w2_ref, b2_ref, o_ref):
    h = _ln(f_ref[...], g_ref[...], b_ref[...])
    h = jnp.maximum(h @ w1_ref[...] + b1_ref[...], 0.0)
    o_ref[...] = jnp.maximum(h @ w2_ref[...] + b2_ref[...], 0.0)


def _embed(feat, g, b, w1, b1, w2, b2):
    f = feat.shape[1]
    fixed = lambda *blk: pl.BlockSpec(blk, lambda i: (0,) * len(blk))
    return pl.pallas_call(
        _embed_body,
        grid=(N // NB,),
        in_specs=[
            pl.BlockSpec((NB, f), lambda i: (i, 0)),
            fixed(1, f), fixed(1, f), fixed(f, EMB), fixed(1, EMB),
            fixed(EMB, EMB), fixed(1, EMB),
        ],
        out_specs=pl.BlockSpec((NB, EMB), lambda i: (i, 0)),
        out_shape=jax.ShapeDtypeStruct((N, EMB), _f32),
    )(feat, g.reshape(1, f), b.reshape(1, f), w1, b1.reshape(1, EMB),
      w2, b2.reshape(1, EMB))


def _pre_body(r_ref, l_ref, wl_ref, bl_ref, eb_ref, we_ref, wr_ref,
              rw_ref, lw_ref):
    bias = bl_ref[...] + eb_ref[...] * we_ref[...]
    rw_ref[...] = r_ref[...] @ wl_ref[...] + bias
    lw_ref[...] = l_ref[...] @ wr_ref[...]


def _node_pre(right, left, pp, ee_b):
    fixed = lambda *blk: pl.BlockSpec(blk, lambda i: (0,) * len(blk))
    node = pl.BlockSpec((NB, EMB), lambda i: (i, 0))
    return pl.pallas_call(
        _pre_body,
        grid=(N // NB,),
        in_specs=[node, node, fixed(EMB, EMB), fixed(1, EMB), fixed(1, 1),
                  fixed(1, EMB), fixed(EMB, EMB)],
        out_specs=[node, node],
        out_shape=[jax.ShapeDtypeStruct((N, EMB), _f32)] * 2,
    )(right, left, pp['Wl'], pp['bl'].reshape(1, EMB), ee_b.reshape(1, 1),
      pp['We'].reshape(1, EMB), pp['Wr'])


def _edgeln_body(s_ref, fg_ref, fb_ref, o_ref):
    o_ref[...] = jnp.maximum(_ln(s_ref[...], fg_ref[...], fb_ref[...]), 0.0)


def _edge_ln(s, fg, fb):
    fixed = lambda *blk: pl.BlockSpec(blk, lambda i: (0,) * len(blk))
    edge = pl.BlockSpec((EB, EMB), lambda i: (i, 0))
    return pl.pallas_call(
        _edgeln_body,
        grid=(EP // EB,),
        in_specs=[edge, fixed(1, EMB), fixed(1, EMB)],
        out_specs=edge,
        out_shape=jax.ShapeDtypeStruct((EP, EMB), _f32),
    )(s, fg.reshape(1, EMB), fb.reshape(1, EMB))


def _post_body(a0_ref, a1_ref, d0_ref, d1_ref, r_ref, wf_ref, bf_ref,
               pg_ref, pb_ref, w1a_ref, w1b_ref, b1_ref, w2_ref, b2_ref,
               o_ref):
    deg = d0_ref[0][:, :1] + d1_ref[0][:, :1]
    agg = (a0_ref[0] + a1_ref[0]) @ wf_ref[...] + deg * bf_ref[...]
    h = _ln(agg, pg_ref[...], pb_ref[...])
    t = jnp.maximum(h @ w1a_ref[...] + r_ref[...] @ w1b_ref[...] + b1_ref[...],
                    0.0)
    o_ref[...] = t @ w2_ref[...] + b2_ref[...]


def _node_post(acc, deg, right, pp):
    fixed = lambda *blk: pl.BlockSpec(blk, lambda i: (0,) * len(blk))
    node = pl.BlockSpec((NB, EMB), lambda i: (i, 0))
    return pl.pallas_call(
        _post_body,
        grid=(N // NB,),
        in_specs=[
            pl.BlockSpec((1, NB, EMB), lambda i: (0, i, 0)),
            pl.BlockSpec((1, NB, EMB), lambda i: (1, i, 0)),
            pl.BlockSpec((1, NB, 16), lambda i: (0, i, 0)),
            pl.BlockSpec((1, NB, 16), lambda i: (1, i, 0)),
            node,
            fixed(EMB, EMB), fixed(1, EMB), fixed(1, EMB), fixed(1, EMB),
            fixed(EMB, EMB), fixed(EMB, EMB), fixed(1, EMB),
            fixed(EMB, EMB), fixed(1, EMB),
        ],
        out_specs=node,
        out_shape=jax.ShapeDtypeStruct((N, EMB), _f32),
    )(acc, acc, deg, deg, right,
      pp['Wf'], pp['bf'].reshape(1, EMB), pp['pg'].reshape(1, EMB),
      pp['pb'].reshape(1, EMB), pp['Wo1'][:EMB], pp['Wo1'][EMB:],
      pp['bo1'].reshape(1, EMB), pp['Wo2'], pp['bo2'].reshape(1, EMB))


def _mlp_body(x_ref, w1_ref, b1_ref, w2_ref, b2_ref, w3_ref, b3_ref, o_ref):
    h = jnp.maximum(x_ref[...] @ w1_ref[...] + b1_ref[...], 0.0)
    h = jnp.maximum(h @ w2_ref[...] + b2_ref[...], 0.0)
    o_ref[...] = h @ w3_ref[...] + b3_ref[...]


def _mlp3(x, pp):
    fixed = lambda *blk: pl.BlockSpec(blk, lambda i: (0,) * len(blk))
    return pl.pallas_call(
        _mlp_body,
        grid=(N // NB,),
        in_specs=[pl.BlockSpec((NB, EMB), lambda i: (i, 0)),
                  fixed(EMB, EMB), fixed(1, EMB), fixed(EMB, EMB),
                  fixed(1, EMB), fixed(EMB, 16), fixed(1, 16)],
        out_specs=pl.BlockSpec((NB, 16), lambda i: (i, 0)),
        out_shape=jax.ShapeDtypeStruct((N, 16), _f32),
    )(x, pp['W1'], pp['b1'].reshape(1, EMB), pp['W2'],
      pp['b2'].reshape(1, EMB), pp['W3'], pp['b3'].reshape(1, 16))


_SC_MESH = plsc.VectorSubcoreMesh(core_axis_name="c", subcore_axis_name="s")


CIB = 2
CROWS = CIB * IW
NCH = IR // 2 // 16 // CIB


@functools.partial(
    pl.kernel,
    out_type=jax.ShapeDtypeStruct((EP, EMB), _f32),
    mesh=_SC_MESH,
    compiler_params=pltpu.CompilerParams(use_tc_tiling_on_sc=False,
                                         needs_layout_passes=False),
    scratch_types=[
        pltpu.VMEM((CIB, IW), jnp.int32),
        pltpu.VMEM((CIB, IW), jnp.int32),
        pltpu.VMEM((CIB, IW), jnp.int32),
        pltpu.VMEM((CIB, IW), jnp.int32),
        pltpu.VMEM((CROWS, EMB), _f32),
        pltpu.VMEM((CROWS, EMB), _f32),
        pltpu.VMEM((CROWS, EMB), _f32),
        pltpu.VMEM((CROWS, EMB), _f32),
        pltpu.VMEM((2, EMB), _f32),
        pltpu.SemaphoreType.DMA,
        pltpu.SemaphoreType.DMA,
        pltpu.SemaphoreType.DMA,
        pltpu.SemaphoreType.DMA,
    ],
)
def _edge_gather(ta_hbm, tb_hbm, ia_hbm, ib_hbm, fgfb_hbm, o_hbm,
                 ixa0, ixa1, ixb0, ixb1, ra0, ra1, rb0, rb1, fgfb,
                 sg0, sg1, ss0, ss1):
    cid = lax.axis_index("c")
    sid = lax.axis_index("s")
    r0 = (cid * 16 + sid) * (NCH * CIB)
    pltpu.sync_copy(fgfb_hbm, fgfb)
    fgl = [fgfb[0, pl.ds(k * 16, 16)] for k in range(4)]
    fbl = [fgfb[1, pl.ds(k * 16, 16)] for k in range(4)]

    def load_idx(rb, ixa, ixb):
        pltpu.sync_copy(ia_hbm.at[pl.ds(rb, CIB)], ixa)
        pltpu.sync_copy(ib_hbm.at[pl.ds(rb, CIB)], ixb)

    def issue_gathers(ixa, ixb, ra, rbuf, sem):
        for j in range(CIB):
            pltpu.async_copy(ta_hbm.at[ixa.at[j]],
                             ra.at[pl.ds(j * IW, IW)], sem)
            pltpu.async_copy(tb_hbm.at[ixb.at[j]],
                             rbuf.at[pl.ds(j * IW, IW)], sem)

    def drain_gathers(ra, rbuf, sem):
        pltpu.make_async_copy(ta_hbm.at[pl.ds(0, CROWS)], ra, sem).wait()
        pltpu.make_async_copy(ta_hbm.at[pl.ds(0, CROWS)], rbuf, sem).wait()

    def add_rows(ra, rbuf):
        inv = 1.0 / EMB

        def abody(r2, carry):
            for u in range(4):
                r = r2 * 4 + u
                s = [ra[r, pl.ds(k * 16, 16)] + rbuf[r, pl.ds(k * 16, 16)]
                     for k in range(4)]
                t = (s[0] + s[1]) + (s[2] + s[3])
                sm = jnp.sum(t) * inv
                mean_v = jnp.broadcast_to(sm, (16,))
                d = [s[k] - mean_v for k in range(4)]
                q = (d[0] * d[0] + d[1] * d[1]) + (d[2] * d[2] + d[3] * d[3])
                var_v = jnp.broadcast_to(jnp.sum(q) * inv + 1e-5, (16,))
                seed = jnp.int32(0x5F3759DF) - (
                    plsc.bitcast(var_v, jnp.int32) >> 1)
                y = plsc.bitcast(seed, _f32)
                xh = var_v * 0.5
                y = y * (1.5 - xh * y * y)
                y = y * (1.5 - xh * y * y)
                y = y * (1.5 - xh * y * y)
                for k in range(4):
                    o = d[k] * (y * fgl[k]) + fbl[k]
                    ra[r, pl.ds(k * 16, 16)] = jnp.maximum(o, 0.0)
            return carry
        lax.fori_loop(0, CROWS // 4, abody, 0)

    def drain_store(ra, sem):
        pltpu.make_async_copy(ra, o_hbm.at[pl.ds(0, CROWS)], sem).wait()

    load_idx(r0, ixa0, ixb0)
    issue_gathers(ixa0, ixb0, ra0, rb0, sg0)

    def body(i, carry):
        c0 = r0 + 2 * i * CIB
        load_idx(c0 + CIB, ixa1, ixb1)

        @pl.when(i >= 1)
        def _():
            drain_store(ra1, ss1)
        issue_gathers(ixa1, ixb1, ra1, rb1, sg1)
        drain_gathers(ra0, rb0, sg0)
        add_rows(ra0, rb0)
        pltpu.async_copy(ra0, o_hbm.at[pl.ds(c0 * IW, CROWS)], ss0)

        @pl.when(i < NCH // 2 - 1)
        def _():
            load_idx(c0 + 2 * CIB, ixa0, ixb0)
            drain_store(ra0, ss0)
            issue_gathers(ixa0, ixb0, ra0, rb0, sg0)
        drain_gathers(ra1, rb1, sg1)
        add_rows(ra1, rb1)
        pltpu.async_copy(ra1, o_hbm.at[pl.ds((c0 + CIB) * IW, CROWS)], ss1)
        return carry

    lax.fori_loop(0, NCH // 2, body, 0)
    drain_store(ra0, ss0)
    drain_store(ra1, ss1)


NSCH = IR // 2 // 16


@functools.partial(
    pl.kernel,
    out_type=jax.ShapeDtypeStruct((2, N, EMB), _f32),
    mesh=_SC_MESH,
    compiler_params=pltpu.CompilerParams(use_tc_tiling_on_sc=False),
    scratch_types=[
        pltpu.VMEM((1, IW), jnp.int32),
        pltpu.VMEM((1, IW), jnp.int32),
        pltpu.VMEM((IW, EMB), _f32),
        pltpu.VMEM((IW, EMB), _f32),
        pltpu.VMEM_SHARED((NACC, EMB), _f32),
        pltpu.SemaphoreType.DMA,
        pltpu.SemaphoreType.DMA,
    ],
)
def _edge_scatter(x_hbm, idx_hbm, z_hbm, out_hbm,
                  ix0, ix1, x0, x1, acc, sx0, sx1):
    cid = lax.axis_index("c")
    sid = lax.axis_index("s")

    pltpu.sync_copy(z_hbm, x0)
    zbase = lax.min(sid * 1600, NACC - 14 * IW)
    for j in range(14):
        pltpu.sync_copy(x0, acc.at[pl.ds(zbase + j * IW, IW)])
    plsc.subcore_barrier()

    r0 = (cid * 16 + sid) * NSCH

    def load(r, ix, xb, sem):
        pltpu.sync_copy(idx_hbm.at[pl.ds(r, 1)], ix)
        pltpu.async_copy(x_hbm.at[pl.ds(r * IW, IW)], xb, sem)

    def scat(ix, xb, sem):
        pltpu.make_async_copy(x_hbm.at[pl.ds(0, IW)], xb, sem).wait()
        pltpu.sync_copy(xb, acc.at[ix.at[0]], add=True)

    load(r0, ix0, x0, sx0)

    def body(i, carry):
        c0 = r0 + 2 * i
        load(c0 + 1, ix1, x1, sx1)
        scat(ix0, x0, sx0)

        @pl.when(i < NSCH // 2 - 1)
        def _():
            load(c0 + 2, ix0, x0, sx0)
        scat(ix1, x1, sx1)
        return carry

    lax.fori_loop(0, NSCH // 2, body, 0)
    plsc.subcore_barrier()

    @pl.when(sid == 0)
    def _():
        pltpu.sync_copy(acc.at[pl.ds(0, N)], out_hbm.at[cid])


@functools.partial(
    pl.kernel,
    out_type=[jax.ShapeDtypeStruct((2, N, 16), _f32)] * 2,
    mesh=_SC_MESH,
    compiler_params=pltpu.CompilerParams(use_tc_tiling_on_sc=False),
    scratch_types=[
        pltpu.VMEM((IB, IW), jnp.int32),
        pltpu.VMEM((IB, IW), jnp.int32),
        pltpu.VMEM((IW, 16), _f32),
        pltpu.VMEM_SHARED((NACC, 16), _f32),
        pltpu.VMEM_SHARED((NACC, 16), _f32),
        pltpu.SemaphoreType.DMA,
        pltpu.SemaphoreType.DMA,
    ],
)
def _degree(i0_hbm, i1_hbm, z_hbm, one_hbm, o0_hbm, o1_hbm,
            ix0, ix1, ones, acc0, acc1, sa, sb):
    cid = lax.axis_index("c")
    sid = lax.axis_index("s")

    pltpu.sync_copy(z_hbm, ones)
    zbase = lax.min(sid * 1600, NACC - 13 * IW)
    for j in range(13):
        pltpu.sync_copy(ones, acc0.at[pl.ds(zbase + j * IW, IW)])
        pltpu.sync_copy(ones, acc1.at[pl.ds(zbase + j * IW, IW)])
    plsc.subcore_barrier()
    pltpu.sync_copy(one_hbm, ones)

    r0 = (cid * 16 + sid) * (NFULL * IB)

    def drain(acc, ix, sem):
        for j in range(IB):
            pltpu.make_async_copy(ones, acc.at[ix.at[j]], sem).wait()

    def outer(k, carry):
        rb = r0 + k * IB

        @pl.when(k >= 1)
        def _():
            drain(acc1, ix1, sb)
        pltpu.sync_copy(i0_hbm.at[pl.ds(rb, IB)], ix0)
        for j in range(IB):
            pltpu.async_copy(ones, acc0.at[ix0.at[j]], sa, add=True)
        pltpu.sync_copy(i1_hbm.at[pl.ds(rb, IB)], ix1)
        drain(acc0, ix0, sa)
        for j in range(IB):
            pltpu.async_copy(ones, acc1.at[ix1.at[j]], sb, add=True)
        return carry

    lax.fori_loop(0, NFULL, outer, 0)
    drain(acc1, ix1, sb)
    plsc.subcore_barrier()

    @pl.when(sid == 0)
    def _():
        pltpu.sync_copy(acc0.at[pl.ds(0, N)], o0_hbm.at[cid])
        pltpu.sync_copy(acc1.at[pl.ds(0, N)], o1_hbm.at[cid])


def _gnn_pass(pp, right, left, dst_g, src_g, dst_s, deg, ee_b, zeros_g):
    rw, lw = _node_pre(right, left, pp, ee_b)
    fgfb = jnp.stack([pp['fg'], pp['fb']])
    x = _edge_gather(rw, lw, dst_g, src_g, fgfb)
    acc = _edge_scatter(x, dst_s, zeros_g)
    return _node_post(acc, deg, right, pp)


def kernel(constraint_features, edge_indices, edge_features,
           variable_features, params):
    p = params
    del edge_features

    pad_g = jnp.arange(NPAD, dtype=jnp.int32) % N
    pad_s = N + (jnp.arange(NPAD, dtype=jnp.int32) % (NACC - N))
    g0 = jnp.concatenate([edge_indices[0], pad_g]).reshape(IR, IW)
    g1 = jnp.concatenate([edge_indices[1], pad_g]).reshape(IR, IW)
    s0 = jnp.concatenate([edge_indices[0], pad_s]).reshape(IR, IW)
    s1 = jnp.concatenate([edge_indices[1], pad_s]).reshape(IR, IW)

    zeros_g = jnp.zeros((IW, EMB), _f32)
    zeros_d = jnp.zeros((IW, 16), _f32)
    ones_d = jnp.ones((IW, 16), _f32)

    c = _embed(constraint_features, p['ce_g'], p['ce_b'], p['ce_W1'],
               p['ce_b1'], p['ce_W2'], p['ce_b2'])
    v = _embed(variable_features, p['ve_g'], p['ve_b'], p['ve_W1'],
               p['ve_b1'], p['ve_W2'], p['ve_b2'])
    deg_c, deg_v = _degree(s0, s1, zeros_d, ones_d)

    for _ in range(3):
        c = _gnn_pass(p['vc'], c, v, g0, g1, s0, deg_c, p['ee_b'], zeros_g)
        v = _gnn_pass(p['cv'], v, c, g1, g0, s1, deg_v, p['ee_b'], zeros_g)

    return _mlp3(c, p['co']), _mlp3(v, p['vo'])
